# baseline (speedup 1.0000x reference)
import functools

import jax
import jax.numpy as jnp
from jax import lax
from jax.experimental import pallas as pl
from jax.experimental.pallas import tpu as pltpu
from jax.experimental.pallas import tpu_sc as plsc

BATCH = 16384
LATENT = 32
ROWS_PER_LINE = 4
LINE = 128
NC = 2
NS = 16
L = 16
NW = NC * NS
BPW = BATCH // NW
CH = 128
NCHK = BPW // CH
GPC = CH // L


def _mf_body(u_idx, i_idx, u_tab, i_tab, out,
             uidx_v, iidx_v, ulid_v, ilid_v, ubuf, ibuf, out_v, sems):
    wid = lax.axis_index("s") * NC + lax.axis_index("c")
    base = wid * BPW

    pltpu.sync_copy(u_idx.at[pl.ds(base, BPW)], uidx_v)
    pltpu.sync_copy(i_idx.at[pl.ds(base, BPW)], iidx_v)

    def to_lines(g, c):
        uv = uidx_v[pl.ds(g * L, L)]
        iv = iidx_v[pl.ds(g * L, L)]
        ulid_v[g // GPC, pl.ds((g % GPC) * L, L)] = lax.shift_right_logical(uv, 2)
        ilid_v[g // GPC, pl.ds((g % GPC) * L, L)] = lax.shift_right_logical(iv, 2)
        return c

    lax.fori_loop(0, BPW // L, to_lines, 0)

    def fire_chunk(k, p):
        pltpu.async_copy(u_tab.at[ulid_v.at[k]], ubuf.at[p], sems.at[0, p])
        pltpu.async_copy(i_tab.at[ilid_v.at[k]], ibuf.at[p], sems.at[1, p])

    def drain_chunk(p):
        pltpu.make_async_copy(
            u_tab.at[pl.ds(0, CH)], ubuf.at[p], sems.at[0, p]).wait()
        pltpu.make_async_copy(
            i_tab.at[pl.ds(0, CH)], ibuf.at[p], sems.at[1, p]).wait()

    fire_chunk(0, 0)
    for k in range(NCHK):
        p = k % 2
        if k + 1 < NCHK:
            fire_chunk(k + 1, (k + 1) % 2)
        drain_chunk(p)

        def group(g, c):
            rows = lax.iota(jnp.int32, L) + g * L
            uv = uidx_v[pl.ds(k * CH + g * L, L)]
            iv = iidx_v[pl.ds(k * CH + g * L, L)]
            ucol = lax.shift_left(lax.bitwise_and(uv, ROWS_PER_LINE - 1), 5)
            icol = lax.shift_left(lax.bitwise_and(iv, ROWS_PER_LINE - 1), 5)
            acc = jnp.zeros((L,), jnp.float32)
            for d in range(LATENT):
                uu = plsc.load_gather(ubuf.at[p], [rows, ucol + d])
                ii = plsc.load_gather(ibuf.at[p], [rows, icol + d])
                acc = acc + uu * ii
            out_v[pl.ds(k * CH + g * L, L)] = acc
            return c

        lax.fori_loop(0, GPC, group, 0)

    pltpu.sync_copy(out_v, out.at[pl.ds(base, BPW)])


@functools.partial(jax.jit)
def kernel(u_idx, i_idx, u_emb, i_emb):
    mesh = plsc.VectorSubcoreMesh(core_axis_name="c", subcore_axis_name="s")
    f = pl.kernel(
        _mf_body,
        out_type=jax.ShapeDtypeStruct((BATCH,), jnp.float32),
        mesh=mesh,
        scratch_types=[
            pltpu.VMEM((BPW,), jnp.int32),
            pltpu.VMEM((BPW,), jnp.int32),
            pltpu.VMEM((NCHK, CH), jnp.int32),
            pltpu.VMEM((NCHK, CH), jnp.int32),
            pltpu.VMEM((2, CH, LINE), jnp.float32),
            pltpu.VMEM((2, CH, LINE), jnp.float32),
            pltpu.VMEM((BPW,), jnp.float32),
            pltpu.SemaphoreType.DMA((2, 2)),
        ],
        compiler_params=pltpu.CompilerParams(needs_layout_passes=False),
    )
    n_lines = u_emb.shape[0] * LATENT // LINE
    u_tab = u_emb.reshape(n_lines, LINE)
    i_tab = i_emb.reshape(n_lines, LINE)
    return f(u_idx, i_idx, u_tab, i_tab)


# speedup vs baseline: 1.4975x; 1.4975x over previous
import functools

import jax
import jax.numpy as jnp
from jax import lax
from jax.experimental import pallas as pl
from jax.experimental.pallas import tpu as pltpu
from jax.experimental.pallas import tpu_sc as plsc

BATCH = 16384
LATENT = 32
NC = 2
NS = 16
L = 16
NW = NC * NS
BPW = BATCH // NW
CH = 128
NCHK = BPW // CH
GPC = CH // L


def _mf_body(u_idx, i_idx, u_emb, i_emb, out,
             uidx_v, iidx_v, ubuf, ibuf, out_v, sems):
    wid = lax.axis_index("s") * NC + lax.axis_index("c")
    base = wid * BPW

    pltpu.sync_copy(u_idx.at[pl.ds(base, BPW)], uidx_v)
    pltpu.sync_copy(i_idx.at[pl.ds(base, BPW)], iidx_v)

    def fire_chunk(k, p):
        def fire(g, c):
            uvec = uidx_v[pl.ds(k * CH + g * L, L)]
            ivec = iidx_v[pl.ds(k * CH + g * L, L)]
            for j in range(L):
                b = g * L + j
                pltpu.async_copy(
                    u_emb.at[pl.ds(uvec[j], 1)],
                    ubuf.at[p].at[pl.ds(b, 1)], sems.at[0, p])
                pltpu.async_copy(
                    i_emb.at[pl.ds(ivec[j], 1)],
                    ibuf.at[p].at[pl.ds(b, 1)], sems.at[1, p])
            return c
        lax.fori_loop(0, GPC, fire, 0)

    def drain_chunk(p):
        pltpu.make_async_copy(
            u_emb.at[pl.ds(0, CH)], ubuf.at[p], sems.at[0, p]).wait()
        pltpu.make_async_copy(
            i_emb.at[pl.ds(0, CH)], ibuf.at[p], sems.at[1, p]).wait()

    fire_chunk(0, 0)
    for k in range(NCHK):
        p = k % 2
        if k + 1 < NCHK:
            fire_chunk(k + 1, (k + 1) % 2)
        drain_chunk(p)

        def group(g, c):
            rows = lax.iota(jnp.int32, L) + g * L
            acc = jnp.zeros((L,), jnp.float32)
            for d in range(LATENT):
                cols = jnp.full((L,), d, jnp.int32)
                uu = plsc.load_gather(ubuf.at[p], [rows, cols])
                ii = plsc.load_gather(ibuf.at[p], [rows, cols])
                acc = acc + uu * ii
            out_v[pl.ds(k * CH + g * L, L)] = acc
            return c

        lax.fori_loop(0, GPC, group, 0)

    pltpu.sync_copy(out_v, out.at[pl.ds(base, BPW)])


@functools.partial(jax.jit)
def kernel(u_idx, i_idx, u_emb, i_emb):
    mesh = plsc.VectorSubcoreMesh(core_axis_name="c", subcore_axis_name="s")
    f = pl.kernel(
        _mf_body,
        out_type=jax.ShapeDtypeStruct((BATCH,), jnp.float32),
        mesh=mesh,
        scratch_types=[
            pltpu.VMEM((BPW,), jnp.int32),
            pltpu.VMEM((BPW,), jnp.int32),
            pltpu.VMEM((2, CH, LATENT), jnp.float32),
            pltpu.VMEM((2, CH, LATENT), jnp.float32),
            pltpu.VMEM((BPW,), jnp.float32),
            pltpu.SemaphoreType.DMA((2, 2)),
        ],
        compiler_params=pltpu.CompilerParams(needs_layout_passes=False),
    )
    return f(u_idx, i_idx, u_emb, i_emb)
